# baseline (speedup 1.0000x reference)
import functools
import math

import numpy as np
import jax
import jax.numpy as jnp
from jax import lax
from jax.experimental import pallas as pl
from jax.experimental.pallas import tpu as pltpu
from jax.experimental.pallas import tpu_sc as plsc

VOCAB = 1000000
D = 64
ROWS = 4096
COLS = 200
SCALE = math.sqrt(D)

NC = 2
NS = 16
NW = NC * NS
L = 16

VCH = 128
NFULL = VOCAB // VCH
VTAIL = VOCAB - NFULL * VCH
TAIL_START = NFULL * VCH
PAIR_ROWS = TAIL_START // 2
FULL_PER_W = NFULL // NW
EXTRA_FULL = NFULL - FULL_PER_W * NW


VC = 1536
NTC = TAIL_START // VC


def _tc_tr_body(in_ref, out_ref):
    t = in_ref[...].T
    out_ref[...] = t.reshape(VC // 2, 2 * D) * SCALE


@jax.jit
def _transpose_scale_tc(lutT):
    return pl.pallas_call(
        _tc_tr_body,
        grid=(NTC,),
        in_specs=[pl.BlockSpec((D, VC), lambda g: (0, g))],
        out_specs=pl.BlockSpec((VC // 2, 2 * D), lambda g: (g, 0)),
        out_shape=jax.ShapeDtypeStruct((PAIR_ROWS, 2 * D), jnp.float32),
        compiler_params=pltpu.CompilerParams(
            dimension_semantics=("arbitrary",)),
    )(lutT)


def _tr_body(lutT, pair_out, t0, t1, w0, w1, s0, s1, so0, so1):
    wid = lax.axis_index("s") * NC + lax.axis_index("c")
    tbuf = (t0, t1)
    wbuf = (w0, w1)
    sin = (s0, s1)
    sout = (so0, so1)
    iota = lax.iota(jnp.int32, L)
    pbase = lax.shift_right_logical(iota, 1)
    qpar = lax.bitwise_and(iota, 1) * D

    def start_in(j, b):
        pltpu.async_copy(lutT.at[:, pl.ds(j * VCH, VCH)], tbuf[b], sin[b])

    def transpose_chunk(b):
        @plsc.parallel_loop(0, D, unroll=2)
        def _(d):
            for jb in range(VCH // L):
                v = tbuf[b][d, pl.ds(jb * L, L)]
                plsc.store_scatter(wbuf[b], [pbase + jb * (L // 2), qpar + d],
                                   v * SCALE)

    def step(j, b, i):
        pltpu.make_async_copy(lutT.at[:, pl.ds(0, VCH)], tbuf[b], sin[b]).wait()

        @pl.when(i >= 2)
        def _():
            pltpu.make_async_copy(w0, pair_out.at[pl.ds(0, VCH // 2)], sout[b]).wait()

        transpose_chunk(b)
        pltpu.async_copy(wbuf[b], pair_out.at[pl.ds(j * (VCH // 2), VCH // 2)],
                         sout[b])

    start_in(wid, 0)
    start_in(wid + NW, 1)

    def pair_iter(t, carry):
        step(wid + NW * (2 * t), 0, 2 * t)

        @pl.when(2 * t + 2 < FULL_PER_W)
        def _():
            start_in(wid + NW * (2 * t + 2), 0)

        step(wid + NW * (2 * t + 1), 1, 2 * t + 1)

        @pl.when(2 * t + 3 < FULL_PER_W)
        def _():
            start_in(wid + NW * (2 * t + 3), 1)

        return carry

    lax.fori_loop(0, FULL_PER_W // 2, pair_iter, 0)

    pltpu.make_async_copy(w0, pair_out.at[pl.ds(0, VCH // 2)], sout[0]).wait()
    pltpu.make_async_copy(w1, pair_out.at[pl.ds(0, VCH // 2)], sout[1]).wait()

    @pl.when(wid < EXTRA_FULL)
    def _():
        j = NW * FULL_PER_W + wid
        pltpu.sync_copy(lutT.at[:, pl.ds(j * VCH, VCH)], t0)
        transpose_chunk(0)
        pltpu.sync_copy(w0, pair_out.at[pl.ds(j * (VCH // 2), VCH // 2)])


@jax.jit
def _transpose_scale(lutT):
    mesh = plsc.VectorSubcoreMesh(core_axis_name="c", subcore_axis_name="s")
    kern = functools.partial(
        pl.kernel,
        mesh=mesh,
        out_type=jax.ShapeDtypeStruct((PAIR_ROWS, 2 * D), jnp.float32),
        scratch_types=[
            pltpu.VMEM((D, VCH), jnp.float32),
            pltpu.VMEM((D, VCH), jnp.float32),
            pltpu.VMEM((VCH // 2, 2 * D), jnp.float32),
            pltpu.VMEM((VCH // 2, 2 * D), jnp.float32),
            pltpu.SemaphoreType.DMA,
            pltpu.SemaphoreType.DMA,
            pltpu.SemaphoreType.DMA,
            pltpu.SemaphoreType.DMA,
        ],
        compiler_params=pltpu.CompilerParams(use_tc_tiling_on_sc=True,
                                             needs_layout_passes=False),
    )(_tr_body)
    return kern(lutT)


CHB = 256
CW = 65
BLK_PER_C = ROWS // CHB
NCHUNK = COLS * BLK_PER_C
CH_PER_W = NCHUNK // NW


def _gat_body(xf, pairs, tail_lin, outP,
              ir0, ir1, ih0, ih1, g0, g1, cb, o0, o1, tpair,
              sgi0, sgi1, sg0, sg1, so0, so1):
    wid = lax.axis_index("s") * NC + lax.axis_index("c")
    iraw = (ir0, ir1)
    ihlf = (ih0, ih1)
    gbuf = (g0, g1)
    obuf = (o0, o1)
    sgi = (sgi0, sgi1)
    sg = (sg0, sg1)
    so = (so0, so1)
    iota = lax.iota(jnp.int32, L)
    iota64 = iota * D

    pltpu.sync_copy(tail_lin, tpair)

    def start_idx(q, b):
        pltpu.async_copy(xf.at[pl.ds(q * CHB, CHB)], iraw[b].at[pl.ds(0, CHB)],
                         sgi[b])

    def wait_idx(b):
        pltpu.make_async_copy(xf.at[pl.ds(0, CHB)], iraw[b].at[pl.ds(0, CHB)],
                              sgi[b]).wait()

    def start_gather(b):
        @plsc.parallel_loop(0, CHB // L, unroll=4)
        def _(g):
            sl = pl.ds(g * L, L)
            h = lax.shift_right_logical(iraw[b][sl], 1)
            ihlf[b][sl] = lax.min(h, jnp.full((L,), PAIR_ROWS - 1, jnp.int32))
        pltpu.async_copy(pairs.at[ihlf[b]], gbuf[b], sg[b])

    def step(q, b, i):
        pltpu.make_async_copy(pairs.at[ihlf[b]], gbuf[b], sg[b]).wait()

        @pl.when(i >= 2)
        def _():
            pltpu.make_async_copy(o0, outP.at[0, :, pl.ds(0, CHB)], so[b]).wait()

        @plsc.parallel_loop(0, CHB, unroll=2)
        def _(l):
            s = iraw[b][pl.ds(l, L)][0]
            rowl = jnp.full((L,), 0, jnp.int32) + l
            colb = iota + (s & 1) * D
            for dblk in range(D // L):
                v = plsc.load_gather(gbuf[b], [rowl, colb + dblk * L])
                cb[pl.ds(l * CW + dblk * L, L)] = v

        @plsc.parallel_loop(0, CHB // L)
        def _(g):
            sl = pl.ds(g * L, L)
            rowv = iota * CW + g * (L * CW)
            for d in range(D):
                v = plsc.load_gather(cb, [rowv + d])
                obuf[b][d, sl] = v * SCALE

        def red(g, acc):
            return lax.max(acc, iraw[b][pl.ds(g * L, L)])

        mx = lax.fori_loop(0, CHB // L, red,
                           jnp.full((L,), 0, jnp.int32), unroll=4)

        @pl.when(jnp.max(mx) >= TAIL_START)
        def _():
            def patch(g, carry):
                sl = pl.ds(g * L, L)
                idxv = iraw[b][sl]
                is_tail = idxv >= TAIL_START
                par = lax.bitwise_and(idxv, 1)
                colbase = par * D
                rowt = lax.max(
                    lax.min(lax.shift_right_logical(idxv, 1) - PAIR_ROWS,
                            jnp.full((L,), VTAIL // 2 - 1, jnp.int32)),
                    jnp.full((L,), 0, jnp.int32))
                for d in range(D):
                    vt = plsc.load_gather(tpair, [rowt, colbase + d])
                    vm = obuf[b][d, sl]
                    obuf[b][d, sl] = lax.select(is_tail, vt * SCALE, vm)
                return carry

            lax.fori_loop(0, CHB // L, patch, 0)

        c = q // BLK_PER_C
        b0 = (q % BLK_PER_C) * CHB
        pltpu.async_copy(obuf[b], outP.at[c, :, pl.ds(b0, CHB)], so[b])

    def chunk_id(i):
        return wid + NW * i

    start_idx(chunk_id(0), 0)
    start_idx(chunk_id(1), 1)
    wait_idx(0)
    start_gather(0)
    wait_idx(1)
    start_gather(1)

    def pair_iter(t, carry):
        step(chunk_id(2 * t), 0, 2 * t)

        @pl.when(2 * t + 2 < CH_PER_W)
        def _():
            start_idx(chunk_id(2 * t + 2), 0)
            wait_idx(0)
            start_gather(0)

        step(chunk_id(2 * t + 1), 1, 2 * t + 1)

        @pl.when(2 * t + 3 < CH_PER_W)
        def _():
            start_idx(chunk_id(2 * t + 3), 1)
            wait_idx(1)
            start_gather(1)

        return carry

    lax.fori_loop(0, CH_PER_W // 2, pair_iter, 0)

    pltpu.make_async_copy(o0, outP.at[0, :, pl.ds(0, CHB)], so[0]).wait()
    pltpu.make_async_copy(o1, outP.at[0, :, pl.ds(0, CHB)], so[1]).wait()


@jax.jit
def _gather(xf, pairs, tail_lin):
    mesh = plsc.VectorSubcoreMesh(core_axis_name="c", subcore_axis_name="s")
    kern = functools.partial(
        pl.kernel,
        mesh=mesh,
        out_type=jax.ShapeDtypeStruct((COLS, D, ROWS), jnp.float32),
        scratch_types=[
            pltpu.VMEM((CHB + L,), jnp.int32),
            pltpu.VMEM((CHB + L,), jnp.int32),
            pltpu.VMEM((CHB,), jnp.int32),
            pltpu.VMEM((CHB,), jnp.int32),
            pltpu.VMEM((CHB, 2 * D), jnp.float32),
            pltpu.VMEM((CHB, 2 * D), jnp.float32),
            pltpu.VMEM((CHB * CW,), jnp.float32),
            pltpu.VMEM((D, CHB), jnp.float32),
            pltpu.VMEM((D, CHB), jnp.float32),
            pltpu.VMEM((VTAIL // 2, 2 * D), jnp.float32),
            pltpu.SemaphoreType.DMA,
            pltpu.SemaphoreType.DMA,
            pltpu.SemaphoreType.DMA,
            pltpu.SemaphoreType.DMA,
            pltpu.SemaphoreType.DMA,
            pltpu.SemaphoreType.DMA,
        ],
        compiler_params=pltpu.CompilerParams(use_tc_tiling_on_sc=True,
                                             needs_layout_passes=False),
    )(_gat_body)
    return kern(xf, pairs, tail_lin)


def kernel(x, lut):
    lutT = lut.T
    pairs = jnp.reshape(lut[:TAIL_START], (PAIR_ROWS, 2 * D))
    xf = x.T.astype(jnp.int32).reshape(ROWS * COLS)
    tail_lin = lut[TAIL_START:].reshape(VTAIL // 2, 2 * D)
    outP = _gather(xf, pairs, tail_lin)
    return outP.transpose(2, 0, 1)


# speedup vs baseline: 1.0086x; 1.0086x over previous
import functools
import math

import numpy as np
import jax
import jax.numpy as jnp
from jax import lax
from jax.experimental import pallas as pl
from jax.experimental.pallas import tpu as pltpu
from jax.experimental.pallas import tpu_sc as plsc

VOCAB = 1000000
D = 64
ROWS = 4096
COLS = 200
SCALE = math.sqrt(D)

NC = 2
NS = 16
NW = NC * NS
L = 16

VCH = 128
NFULL = VOCAB // VCH
VTAIL = VOCAB - NFULL * VCH
TAIL_START = NFULL * VCH
PAIR_ROWS = TAIL_START // 2
FULL_PER_W = NFULL // NW
EXTRA_FULL = NFULL - FULL_PER_W * NW


VC = 1536
NTC = TAIL_START // VC


def _tc_tr_body(in_ref, out_ref):
    t = in_ref[...].T
    out_ref[...] = t.reshape(VC // 2, 2 * D) * SCALE


@jax.jit
def _transpose_scale_tc(lutT):
    return pl.pallas_call(
        _tc_tr_body,
        grid=(NTC,),
        in_specs=[pl.BlockSpec((D, VC), lambda g: (0, g))],
        out_specs=pl.BlockSpec((VC // 2, 2 * D), lambda g: (g, 0)),
        out_shape=jax.ShapeDtypeStruct((PAIR_ROWS, 2 * D), jnp.float32),
        compiler_params=pltpu.CompilerParams(
            dimension_semantics=("arbitrary",)),
    )(lutT)


def _tr_body(lutT, pair_out, t0, t1, w0, w1, s0, s1, so0, so1):
    wid = lax.axis_index("s") * NC + lax.axis_index("c")
    tbuf = (t0, t1)
    wbuf = (w0, w1)
    sin = (s0, s1)
    sout = (so0, so1)
    iota = lax.iota(jnp.int32, L)
    pbase = lax.shift_right_logical(iota, 1)
    qpar = lax.bitwise_and(iota, 1) * D

    def start_in(j, b):
        pltpu.async_copy(lutT.at[:, pl.ds(j * VCH, VCH)], tbuf[b], sin[b])

    def transpose_chunk(b):
        @plsc.parallel_loop(0, D, unroll=2)
        def _(d):
            for jb in range(VCH // L):
                v = tbuf[b][d, pl.ds(jb * L, L)]
                plsc.store_scatter(wbuf[b], [pbase + jb * (L // 2), qpar + d],
                                   v * SCALE)

    def step(j, b, i):
        pltpu.make_async_copy(lutT.at[:, pl.ds(0, VCH)], tbuf[b], sin[b]).wait()

        @pl.when(i >= 2)
        def _():
            pltpu.make_async_copy(w0, pair_out.at[pl.ds(0, VCH // 2)], sout[b]).wait()

        transpose_chunk(b)
        pltpu.async_copy(wbuf[b], pair_out.at[pl.ds(j * (VCH // 2), VCH // 2)],
                         sout[b])

    start_in(wid, 0)
    start_in(wid + NW, 1)

    def pair_iter(t, carry):
        step(wid + NW * (2 * t), 0, 2 * t)

        @pl.when(2 * t + 2 < FULL_PER_W)
        def _():
            start_in(wid + NW * (2 * t + 2), 0)

        step(wid + NW * (2 * t + 1), 1, 2 * t + 1)

        @pl.when(2 * t + 3 < FULL_PER_W)
        def _():
            start_in(wid + NW * (2 * t + 3), 1)

        return carry

    lax.fori_loop(0, FULL_PER_W // 2, pair_iter, 0)

    pltpu.make_async_copy(w0, pair_out.at[pl.ds(0, VCH // 2)], sout[0]).wait()
    pltpu.make_async_copy(w1, pair_out.at[pl.ds(0, VCH // 2)], sout[1]).wait()

    @pl.when(wid < EXTRA_FULL)
    def _():
        j = NW * FULL_PER_W + wid
        pltpu.sync_copy(lutT.at[:, pl.ds(j * VCH, VCH)], t0)
        transpose_chunk(0)
        pltpu.sync_copy(w0, pair_out.at[pl.ds(j * (VCH // 2), VCH // 2)])


@jax.jit
def _transpose_scale(lutT):
    mesh = plsc.VectorSubcoreMesh(core_axis_name="c", subcore_axis_name="s")
    kern = functools.partial(
        pl.kernel,
        mesh=mesh,
        out_type=jax.ShapeDtypeStruct((PAIR_ROWS, 2 * D), jnp.float32),
        scratch_types=[
            pltpu.VMEM((D, VCH), jnp.float32),
            pltpu.VMEM((D, VCH), jnp.float32),
            pltpu.VMEM((VCH // 2, 2 * D), jnp.float32),
            pltpu.VMEM((VCH // 2, 2 * D), jnp.float32),
            pltpu.SemaphoreType.DMA,
            pltpu.SemaphoreType.DMA,
            pltpu.SemaphoreType.DMA,
            pltpu.SemaphoreType.DMA,
        ],
        compiler_params=pltpu.CompilerParams(use_tc_tiling_on_sc=True,
                                             needs_layout_passes=False),
    )(_tr_body)
    return kern(lutT)


CHB = 256
CW = 65
BLK_PER_C = ROWS // CHB
NCHUNK = COLS * BLK_PER_C
CH_PER_W = NCHUNK // NW


def _gat_body(xf, pairs, tail_lin, outP,
              ir0, ir1, ih0, ih1, g0, g1, cb, o0, o1, tpair,
              sgi0, sgi1, sg0, sg1, so0, so1):
    wid = lax.axis_index("s") * NC + lax.axis_index("c")
    iraw = (ir0, ir1)
    ihlf = (ih0, ih1)
    gbuf = (g0, g1)
    obuf = (o0, o1)
    sgi = (sgi0, sgi1)
    sg = (sg0, sg1)
    so = (so0, so1)
    iota = lax.iota(jnp.int32, L)
    iota64 = iota * D

    pltpu.sync_copy(tail_lin, tpair)

    def start_idx(q, b):
        pltpu.async_copy(xf.at[pl.ds(q * CHB, CHB)], iraw[b].at[pl.ds(0, CHB)],
                         sgi[b])

    def wait_idx(b):
        pltpu.make_async_copy(xf.at[pl.ds(0, CHB)], iraw[b].at[pl.ds(0, CHB)],
                              sgi[b]).wait()

    def start_gather(b):
        @plsc.parallel_loop(0, CHB // L, unroll=4)
        def _(g):
            sl = pl.ds(g * L, L)
            h = lax.shift_right_logical(iraw[b][sl], 1)
            ihlf[b][sl] = lax.min(h, jnp.full((L,), PAIR_ROWS - 1, jnp.int32))
        pltpu.async_copy(pairs.at[ihlf[b]], gbuf[b], sg[b])

    def step(q, b, i):
        pltpu.make_async_copy(pairs.at[ihlf[b]], gbuf[b], sg[b]).wait()

        @pl.when(i >= 2)
        def _():
            pltpu.make_async_copy(o0, outP.at[0, :, pl.ds(0, CHB)], so[b]).wait()

        @plsc.parallel_loop(0, CHB, unroll=4)
        def _(l):
            s = iraw[b][pl.ds(l, L)][0]
            rowl = jnp.full((L,), 0, jnp.int32) + l
            colb = iota + (s & 1) * D
            for dblk in range(D // L):
                v = plsc.load_gather(gbuf[b], [rowl, colb + dblk * L])
                cb[pl.ds(l * CW + dblk * L, L)] = v

        @plsc.parallel_loop(0, CHB // L)
        def _(g):
            sl = pl.ds(g * L, L)
            rowv = iota * CW + g * (L * CW)
            for d in range(D):
                v = plsc.load_gather(cb, [rowv + d])
                obuf[b][d, sl] = v * SCALE

        def red(g, acc):
            return lax.max(acc, iraw[b][pl.ds(g * L, L)])

        mx = lax.fori_loop(0, CHB // L, red,
                           jnp.full((L,), 0, jnp.int32), unroll=4)

        @pl.when(jnp.max(mx) >= TAIL_START)
        def _():
            def patch(g, carry):
                sl = pl.ds(g * L, L)
                idxv = iraw[b][sl]
                is_tail = idxv >= TAIL_START
                par = lax.bitwise_and(idxv, 1)
                colbase = par * D
                rowt = lax.max(
                    lax.min(lax.shift_right_logical(idxv, 1) - PAIR_ROWS,
                            jnp.full((L,), VTAIL // 2 - 1, jnp.int32)),
                    jnp.full((L,), 0, jnp.int32))
                for d in range(D):
                    vt = plsc.load_gather(tpair, [rowt, colbase + d])
                    vm = obuf[b][d, sl]
                    obuf[b][d, sl] = lax.select(is_tail, vt * SCALE, vm)
                return carry

            lax.fori_loop(0, CHB // L, patch, 0)

        c = q // BLK_PER_C
        b0 = (q % BLK_PER_C) * CHB
        pltpu.async_copy(obuf[b], outP.at[c, :, pl.ds(b0, CHB)], so[b])

    def chunk_id(i):
        return wid + NW * i

    start_idx(chunk_id(0), 0)
    start_idx(chunk_id(1), 1)
    wait_idx(0)
    start_gather(0)
    wait_idx(1)
    start_gather(1)

    def pair_iter(t, carry):
        step(chunk_id(2 * t), 0, 2 * t)

        @pl.when(2 * t + 2 < CH_PER_W)
        def _():
            start_idx(chunk_id(2 * t + 2), 0)
            wait_idx(0)
            start_gather(0)

        step(chunk_id(2 * t + 1), 1, 2 * t + 1)

        @pl.when(2 * t + 3 < CH_PER_W)
        def _():
            start_idx(chunk_id(2 * t + 3), 1)
            wait_idx(1)
            start_gather(1)

        return carry

    lax.fori_loop(0, CH_PER_W // 2, pair_iter, 0)

    pltpu.make_async_copy(o0, outP.at[0, :, pl.ds(0, CHB)], so[0]).wait()
    pltpu.make_async_copy(o1, outP.at[0, :, pl.ds(0, CHB)], so[1]).wait()


@jax.jit
def _gather(xf, pairs, tail_lin):
    mesh = plsc.VectorSubcoreMesh(core_axis_name="c", subcore_axis_name="s")
    kern = functools.partial(
        pl.kernel,
        mesh=mesh,
        out_type=jax.ShapeDtypeStruct((COLS, D, ROWS), jnp.float32),
        scratch_types=[
            pltpu.VMEM((CHB + L,), jnp.int32),
            pltpu.VMEM((CHB + L,), jnp.int32),
            pltpu.VMEM((CHB,), jnp.int32),
            pltpu.VMEM((CHB,), jnp.int32),
            pltpu.VMEM((CHB, 2 * D), jnp.float32),
            pltpu.VMEM((CHB, 2 * D), jnp.float32),
            pltpu.VMEM((CHB * CW,), jnp.float32),
            pltpu.VMEM((D, CHB), jnp.float32),
            pltpu.VMEM((D, CHB), jnp.float32),
            pltpu.VMEM((VTAIL // 2, 2 * D), jnp.float32),
            pltpu.SemaphoreType.DMA,
            pltpu.SemaphoreType.DMA,
            pltpu.SemaphoreType.DMA,
            pltpu.SemaphoreType.DMA,
            pltpu.SemaphoreType.DMA,
            pltpu.SemaphoreType.DMA,
        ],
        compiler_params=pltpu.CompilerParams(use_tc_tiling_on_sc=True,
                                             needs_layout_passes=False),
    )(_gat_body)
    return kern(xf, pairs, tail_lin)


def kernel(x, lut):
    lutT = lut.T
    pairs = jnp.reshape(lut[:TAIL_START], (PAIR_ROWS, 2 * D))
    xf = x.T.astype(jnp.int32).reshape(ROWS * COLS)
    tail_lin = lut[TAIL_START:].reshape(VTAIL // 2, 2 * D)
    outP = _gather(xf, pairs, tail_lin)
    return outP.transpose(2, 0, 1)


# speedup vs baseline: 1.0116x; 1.0030x over previous
import functools
import math

import jax
import jax.numpy as jnp
from jax import lax
from jax.experimental import pallas as pl
from jax.experimental.pallas import tpu as pltpu
from jax.experimental.pallas import tpu_sc as plsc

VOCAB = 1000000
D = 64
ROWS = 4096
COLS = 200
SCALE = math.sqrt(D)

NC = 2
NS = 16
NW = NC * NS
L = 16

VCH = 128
NFULL = VOCAB // VCH
VTAIL = VOCAB - NFULL * VCH
TAIL_START = NFULL * VCH
PAIR_ROWS = TAIL_START // 2

CHB = 256
CW = 65
BLK_PER_C = ROWS // CHB
NCHUNK = COLS * BLK_PER_C
CH_PER_W = NCHUNK // NW


def _gat_body(xf, pairs, tail_lin, outP,
              ir0, ir1, ih0, ih1, g0, g1, cb, o0, o1, tpair,
              sgi0, sgi1, sg0, sg1, so0, so1):
    wid = lax.axis_index("s") * NC + lax.axis_index("c")
    iraw = (ir0, ir1)
    ihlf = (ih0, ih1)
    gbuf = (g0, g1)
    obuf = (o0, o1)
    sgi = (sgi0, sgi1)
    sg = (sg0, sg1)
    so = (so0, so1)
    iota = lax.iota(jnp.int32, L)

    pltpu.sync_copy(tail_lin, tpair)

    def start_idx(q, b):
        pltpu.async_copy(xf.at[pl.ds(q * CHB, CHB)], iraw[b].at[pl.ds(0, CHB)],
                         sgi[b])

    def wait_idx(b):
        pltpu.make_async_copy(xf.at[pl.ds(0, CHB)], iraw[b].at[pl.ds(0, CHB)],
                              sgi[b]).wait()

    def start_gather(b):
        @plsc.parallel_loop(0, CHB // L, unroll=4)
        def _(g):
            sl = pl.ds(g * L, L)
            h = lax.shift_right_logical(iraw[b][sl], 1)
            ihlf[b][sl] = lax.min(h, jnp.full((L,), PAIR_ROWS - 1, jnp.int32))
        pltpu.async_copy(pairs.at[ihlf[b]], gbuf[b], sg[b])

    def step(q, b, i):
        pltpu.make_async_copy(pairs.at[ihlf[b]], gbuf[b], sg[b]).wait()

        @pl.when(i >= 2)
        def _():
            pltpu.make_async_copy(o0, outP.at[0, :, pl.ds(0, CHB)], so[b]).wait()

        @plsc.parallel_loop(0, CHB, unroll=4)
        def _(l):
            s = iraw[b][pl.ds(l, L)][0]
            rowl = jnp.full((L,), 0, jnp.int32) + l
            colb = iota + (s & 1) * D
            for dblk in range(D // L):
                v = plsc.load_gather(gbuf[b], [rowl, colb + dblk * L])
                cb[pl.ds(l * CW + dblk * L, L)] = v

        @plsc.parallel_loop(0, CHB // L)
        def _(g):
            sl = pl.ds(g * L, L)
            rowv = iota * CW + g * (L * CW)
            for d in range(D):
                v = plsc.load_gather(cb, [rowv + d])
                obuf[b][d, sl] = v * SCALE

        def red(g, acc):
            return lax.max(acc, iraw[b][pl.ds(g * L, L)])

        mx = lax.fori_loop(0, CHB // L, red,
                           jnp.full((L,), 0, jnp.int32), unroll=4)

        @pl.when(jnp.max(mx) >= TAIL_START)
        def _():
            def patch(g, carry):
                sl = pl.ds(g * L, L)
                idxv = iraw[b][sl]
                is_tail = idxv >= TAIL_START
                par = lax.bitwise_and(idxv, 1)
                colbase = par * D
                rowt = lax.max(
                    lax.min(lax.shift_right_logical(idxv, 1) - PAIR_ROWS,
                            jnp.full((L,), VTAIL // 2 - 1, jnp.int32)),
                    jnp.full((L,), 0, jnp.int32))
                for d in range(D):
                    vt = plsc.load_gather(tpair, [rowt, colbase + d])
                    vm = obuf[b][d, sl]
                    obuf[b][d, sl] = lax.select(is_tail, vt * SCALE, vm)
                return carry

            lax.fori_loop(0, CHB // L, patch, 0)

        c = q // BLK_PER_C
        b0 = (q % BLK_PER_C) * CHB
        pltpu.async_copy(obuf[b], outP.at[c, :, pl.ds(b0, CHB)], so[b])

    def chunk_id(i):
        return wid + NW * i

    start_idx(chunk_id(0), 0)
    start_idx(chunk_id(1), 1)
    wait_idx(0)
    start_gather(0)
    wait_idx(1)
    start_gather(1)

    def pair_iter(t, carry):
        step(chunk_id(2 * t), 0, 2 * t)

        @pl.when(2 * t + 2 < CH_PER_W)
        def _():
            start_idx(chunk_id(2 * t + 2), 0)
            wait_idx(0)
            start_gather(0)

        step(chunk_id(2 * t + 1), 1, 2 * t + 1)

        @pl.when(2 * t + 3 < CH_PER_W)
        def _():
            start_idx(chunk_id(2 * t + 3), 1)
            wait_idx(1)
            start_gather(1)

        return carry

    lax.fori_loop(0, CH_PER_W // 2, pair_iter, 0)

    pltpu.make_async_copy(o0, outP.at[0, :, pl.ds(0, CHB)], so[0]).wait()
    pltpu.make_async_copy(o1, outP.at[0, :, pl.ds(0, CHB)], so[1]).wait()


@jax.jit
def _gather(xf, pairs, tail_lin):
    mesh = plsc.VectorSubcoreMesh(core_axis_name="c", subcore_axis_name="s")
    kern = functools.partial(
        pl.kernel,
        mesh=mesh,
        out_type=jax.ShapeDtypeStruct((COLS, D, ROWS), jnp.float32),
        scratch_types=[
            pltpu.VMEM((CHB + L,), jnp.int32),
            pltpu.VMEM((CHB + L,), jnp.int32),
            pltpu.VMEM((CHB,), jnp.int32),
            pltpu.VMEM((CHB,), jnp.int32),
            pltpu.VMEM((CHB, 2 * D), jnp.float32),
            pltpu.VMEM((CHB, 2 * D), jnp.float32),
            pltpu.VMEM((CHB * CW,), jnp.float32),
            pltpu.VMEM((D, CHB), jnp.float32),
            pltpu.VMEM((D, CHB), jnp.float32),
            pltpu.VMEM((VTAIL // 2, 2 * D), jnp.float32),
            pltpu.SemaphoreType.DMA,
            pltpu.SemaphoreType.DMA,
            pltpu.SemaphoreType.DMA,
            pltpu.SemaphoreType.DMA,
            pltpu.SemaphoreType.DMA,
            pltpu.SemaphoreType.DMA,
        ],
        compiler_params=pltpu.CompilerParams(use_tc_tiling_on_sc=True,
                                             needs_layout_passes=False),
    )(_gat_body)
    return kern(xf, pairs, tail_lin)


def kernel(x, lut):
    pairs = jnp.reshape(lut[:TAIL_START], (PAIR_ROWS, 2 * D))
    xf = x.T.astype(jnp.int32).reshape(ROWS * COLS)
    tail_lin = lut[TAIL_START:].reshape(VTAIL // 2, 2 * D)
    outP = _gather(xf, pairs, tail_lin)
    return outP.transpose(2, 0, 1)
